# baseline (speedup 1.0000x reference)
import functools

import jax
import jax.numpy as jnp
from jax import lax
from jax.experimental import pallas as pl
from jax.experimental.pallas import tpu as pltpu
from jax.experimental.pallas import tpu_sc as plsc

_NC = 2
_NS = 16
_NW = _NC * _NS
_CH = 504


def _sc_copy_pair(t, w):
    m, n = t.shape
    full_chunks = m // _CH
    iters = full_chunks // _NW
    assert iters * _NW == full_chunks
    tail_off = full_chunks * _CH
    tail = m - tail_off
    mesh = plsc.VectorSubcoreMesh(core_axis_name="c", subcore_axis_name="s")

    @functools.partial(
        pl.kernel,
        out_type=(
            jax.ShapeDtypeStruct((m, n), t.dtype),
            jax.ShapeDtypeStruct((m, n), w.dtype),
        ),
        mesh=mesh,
        scratch_types=[
            pltpu.VMEM((_CH, n), t.dtype),
            pltpu.VMEM((_CH, n), t.dtype),
            pltpu.SemaphoreType.DMA((2,)),
            pltpu.SemaphoreType.DMA((2,)),
        ],
    )
    def body(t_in, w_in, t_out, w_out, b0, b1, s_in, s_out):
        wid = lax.axis_index("s") * _NC + lax.axis_index("c")
        bufs = (b0, b1)

        def copy_tensor(in_hbm, out_hbm):
            outs = {}
            for k in range(iters):
                b = k % 2
                if k >= 2:
                    outs[k - 2].wait()
                sl = pl.ds((wid + _NW * k) * _CH, _CH)
                pltpu.async_copy(in_hbm.at[sl], bufs[b], s_in.at[b]).wait()
                outs[k] = pltpu.async_copy(bufs[b], out_hbm.at[sl],
                                           s_out.at[b])
            outs[iters - 2].wait()
            outs[iters - 1].wait()
            if tail:
                @pl.when(wid == 0)
                def _():
                    sl = pl.ds(tail_off, tail)
                    bsl = pl.ds(0, tail)
                    pltpu.async_copy(in_hbm.at[sl], b0.at[bsl],
                                     s_in.at[0]).wait()
                    pltpu.async_copy(b0.at[bsl], out_hbm.at[sl],
                                     s_out.at[0]).wait()

        copy_tensor(t_in, t_out)
        copy_tensor(w_in, w_out)

    return body(t, w)


def kernel(bbox_targets, bbox_weights, labels):
    del labels
    return _sc_copy_pair(bbox_targets, bbox_weights)


# speedup vs baseline: 1.0531x; 1.0531x over previous
import jax
from jax.experimental import pallas as pl

_BR = 8000


def _copy_kernel(t_in, w_in, t_out, w_out):
    t_out[...] = t_in[...]
    w_out[...] = w_in[...]


def kernel(bbox_targets, bbox_weights, labels):
    del labels
    m, n = bbox_targets.shape
    grid = m // _BR
    spec = pl.BlockSpec((_BR, n), lambda i: (i, 0))
    out_shape = (
        jax.ShapeDtypeStruct((m, n), bbox_targets.dtype),
        jax.ShapeDtypeStruct((m, n), bbox_weights.dtype),
    )
    t, w = pl.pallas_call(
        _copy_kernel,
        grid=(grid,),
        out_shape=out_shape,
        in_specs=[spec, spec],
        out_specs=[spec, spec],
    )(bbox_targets, bbox_weights)
    return (t, w)
